# baseline (speedup 1.0000x reference)
import functools

import jax
import jax.numpy as jnp
from jax import lax
from jax.experimental import pallas as pl
from jax.experimental.pallas import tpu as pltpu
from jax.experimental.pallas import tpu_sc as plsc

NC = 2
NS = 16
NW = NC * NS

LANES = 16
LG = 4
DEPTH = 2


@functools.lru_cache(maxsize=None)
def _emb_call(n_b: int, n_l: int, d: int):
    assert d == 2 * LANES
    b_per_w = n_b // NW
    assert b_per_w * NW == n_b and b_per_w == 128
    n_g = n_l // (LG * DEPTH) * DEPTH
    assert n_g * LG == n_l

    mesh = plsc.VectorSubcoreMesh(
        core_axis_name="c", subcore_axis_name="s",
        num_cores=NC, num_subcores=NS)

    @functools.partial(
        pl.kernel,
        out_type=jax.ShapeDtypeStruct((n_l, d, n_b), jnp.float32),
        mesh=mesh,
        scratch_types=[
            pltpu.VMEM((n_g, LG * b_per_w), jnp.int32),
            [pltpu.VMEM((LG * b_per_w, d), jnp.float32)] * DEPTH,
            [pltpu.VMEM((LG, d, b_per_w + 1), jnp.float32)] * DEPTH,
            pltpu.VMEM((n_l, d), jnp.float32),
            [pltpu.SemaphoreType.DMA] * DEPTH,
            [pltpu.SemaphoreType.DMA] * DEPTH,
        ],
        compiler_params=pltpu.CompilerParams(
            use_tc_tiling_on_sc=False, needs_layout_passes=False,
            skip_device_barrier=True),
    )
    def run(idx_hbm, table_hbm, pos_hbm, out_hbm,
            idx_v, rows_v, outt_v, pos_v, gsem, osem):
        wid = lax.axis_index("s") * NC + lax.axis_index("c")
        b0 = pl.multiple_of(wid * b_per_w, b_per_w)
        pltpu.sync_copy(pos_hbm, pos_v)
        pltpu.sync_copy(idx_hbm.at[wid], idx_v)

        def fire(g, k):
            pltpu.async_copy(table_hbm.at[idx_v.at[g]], rows_v[k], gsem[k])

        for k in range(DEPTH):
            fire(k, k)

        iota = lax.iota(jnp.int32, LANES)
        row0 = iota
        row1 = iota + LANES

        def handle(g, k):
            l0 = pl.multiple_of(g * LG, LG)
            pltpu.make_async_copy(
                table_hbm.at[idx_v.at[g]], rows_v[k], gsem[k]).wait()
            @pl.when(g >= DEPTH)
            def _():
                pltpu.make_async_copy(
                    outt_v[k].at[:, :, pl.ds(0, b_per_w)],
                    out_hbm.at[pl.ds(l0, LG), :, pl.ds(b0, b_per_w)],
                    osem[k]).wait()

            ps = [(pos_v[l0 + j, pl.ds(0, LANES)],
                   pos_v[l0 + j, pl.ds(LANES, LANES)]) for j in range(LG)]
            src = rows_v[k]
            dst = outt_v[k]

            def tok_body(b, carry):
                col = jnp.full((LANES,), b, jnp.int32)
                for j in range(LG):
                    v0 = src[j * b_per_w + b, pl.ds(0, LANES)] + ps[j][0]
                    v1 = src[j * b_per_w + b, pl.ds(LANES, LANES)] + ps[j][1]
                    plsc.store_scatter(dst.at[j], [row0, col], v0)
                    plsc.store_scatter(dst.at[j], [row1, col], v1)
                return carry

            lax.fori_loop(0, b_per_w, tok_body, 0)

            @pl.when(g < n_g - DEPTH)
            def _():
                fire(g + DEPTH, k)

            pltpu.async_copy(
                dst.at[:, :, pl.ds(0, b_per_w)],
                out_hbm.at[pl.ds(l0, LG), :, pl.ds(b0, b_per_w)],
                osem[k])

        def step(g2, carry):
            for j in range(DEPTH):
                handle(g2 * DEPTH + j, j)
            return carry

        lax.fori_loop(0, n_g // DEPTH, step, 0)

        for k in range(DEPTH):
            pltpu.make_async_copy(
                outt_v[k].at[:, :, pl.ds(0, b_per_w)],
                out_hbm.at[pl.ds(0, LG), :, pl.ds(b0, b_per_w)],
                osem[k]).wait()

    return run


def kernel(input, token_table, pos_table):
    b, l = input.shape
    v, d = token_table.shape
    idx_a = (input.T.astype(jnp.int32)
             .reshape(l // LG, LG, b // 128, 128)
             .transpose(2, 0, 1, 3)
             .reshape(b // 128, l // LG, LG * 128))
    out = _emb_call(b, l, d)(idx_a, token_table,
                             pos_table.astype(jnp.float32))
    return jnp.transpose(out, (2, 0, 1))


# speedup vs baseline: 1.1770x; 1.1770x over previous
import functools

import jax
import jax.numpy as jnp
from jax import lax
from jax.experimental import pallas as pl
from jax.experimental.pallas import tpu as pltpu
from jax.experimental.pallas import tpu_sc as plsc

NC = 2
NS = 16
NW = NC * NS

LANES = 16
LG = 4
DEPTH = 2


@functools.lru_cache(maxsize=None)
def _emb_call(n_b: int, n_l: int, d: int):
    assert d == 2 * LANES
    b_per_w = n_b // NW
    assert b_per_w * NW == n_b and b_per_w == 128
    n_g = n_l // (LG * DEPTH) * DEPTH
    assert n_g * LG == n_l

    mesh = plsc.VectorSubcoreMesh(
        core_axis_name="c", subcore_axis_name="s",
        num_cores=NC, num_subcores=NS)

    @functools.partial(
        pl.kernel,
        out_type=jax.ShapeDtypeStruct(
            (n_l, d // 8, n_b // 128, 8, 128), jnp.float32),
        mesh=mesh,
        scratch_types=[
            pltpu.VMEM((n_g, LG * b_per_w), jnp.int32),
            [pltpu.VMEM((LG * b_per_w, d), jnp.float32)] * DEPTH,
            [pltpu.VMEM((LG, d // 8, 8, b_per_w + 1), jnp.float32)] * DEPTH,
            pltpu.VMEM((n_l, d), jnp.float32),
            [pltpu.SemaphoreType.DMA] * DEPTH,
            [pltpu.SemaphoreType.DMA] * DEPTH,
        ],
        compiler_params=pltpu.CompilerParams(
            use_tc_tiling_on_sc=False, needs_layout_passes=False,
            skip_device_barrier=True),
    )
    def run(idx_hbm, table_hbm, pos_hbm, out_hbm,
            idx_v, rows_v, outt_v, pos_v, gsem, osem):
        wid = lax.axis_index("s") * NC + lax.axis_index("c")
        b0 = pl.multiple_of(wid * b_per_w, b_per_w)
        pltpu.sync_copy(pos_hbm, pos_v)
        pltpu.sync_copy(idx_hbm.at[wid], idx_v)

        def fire(g, k):
            pltpu.async_copy(table_hbm.at[idx_v.at[g]], rows_v[k], gsem[k])

        for k in range(DEPTH):
            fire(k, k)

        iota = lax.iota(jnp.int32, LANES)
        sub0 = lax.rem(iota, 8)
        tr0 = iota // 8
        tr1 = tr0 + 2

        def handle(g, k):
            l0 = pl.multiple_of(g * LG, LG)
            pltpu.make_async_copy(
                table_hbm.at[idx_v.at[g]], rows_v[k], gsem[k]).wait()
            @pl.when(g >= DEPTH)
            def _():
                pltpu.make_async_copy(
                    outt_v[k].at[:, :, :, pl.ds(0, b_per_w)],
                    out_hbm.at[pl.ds(l0, LG), :, wid],
                    osem[k]).wait()

            ps = [(pos_v[l0 + j, pl.ds(0, LANES)],
                   pos_v[l0 + j, pl.ds(LANES, LANES)]) for j in range(LG)]
            src = rows_v[k]
            dst = outt_v[k]

            def tok_body(b, carry):
                col = jnp.full((LANES,), b, jnp.int32)
                for j in range(LG):
                    v0 = src[j * b_per_w + b, pl.ds(0, LANES)] + ps[j][0]
                    v1 = src[j * b_per_w + b, pl.ds(LANES, LANES)] + ps[j][1]
                    plsc.store_scatter(dst.at[j], [tr0, sub0, col], v0)
                    plsc.store_scatter(dst.at[j], [tr1, sub0, col], v1)
                return carry

            lax.fori_loop(0, b_per_w, tok_body, 0)

            @pl.when(g < n_g - DEPTH)
            def _():
                fire(g + DEPTH, k)

            pltpu.async_copy(
                dst.at[:, :, :, pl.ds(0, b_per_w)],
                out_hbm.at[pl.ds(l0, LG), :, wid],
                osem[k])

        def step(g2, carry):
            for j in range(DEPTH):
                handle(g2 * DEPTH + j, j)
            return carry

        lax.fori_loop(0, n_g // DEPTH, step, 0)

        for k in range(DEPTH):
            pltpu.make_async_copy(
                outt_v[k].at[:, :, :, pl.ds(0, b_per_w)],
                out_hbm.at[pl.ds(0, LG), :, wid],
                osem[k]).wait()

    return run


def kernel(input, token_table, pos_table):
    b, l = input.shape
    v, d = token_table.shape
    idx_a = (input.T.astype(jnp.int32)
             .reshape(l // LG, LG, b // 128, 128)
             .transpose(2, 0, 1, 3)
             .reshape(b // 128, l // LG, LG * 128))
    out = _emb_call(b, l, d)(idx_a, token_table,
                             pos_table.astype(jnp.float32))
    return (out.transpose(2, 4, 0, 1, 3)
            .reshape(b, l, d))
